# baseline (speedup 1.0000x reference)
import jax
import jax.numpy as jnp
from jax.experimental import pallas as pl

_B, _N, _D = 256, 2048, 64
_NL = _N // 128
_BG = 4


def _slots_kernel(um_ref, rm_ref, fm_ref, umc_ref, uvec_ref, wvec_ref, e_ref,
                  idx_ref, wm_ref, out_ref, stats_ref):
    i = pl.program_id(0)
    nsteps = pl.num_programs(0)

    e = e_ref[...]
    umf = um_ref[...].astype(jnp.float32)
    rmf = rm_ref[...].astype(jnp.float32)
    fmf = fm_ref[...].astype(jnp.float32)
    wmf = wm_ref[...]

    row = jax.lax.broadcasted_iota(jnp.int32, (_NL, 128), 0)
    col = jax.lax.broadcasted_iota(jnp.int32, (_NL, 128), 1)
    slot = row * 128 + col
    ncol = jax.lax.broadcasted_iota(jnp.int32, (_N, 1), 0)

    aw_parts = []
    for g in range(_BG):
        tgt_g = jnp.clip(idx_ref[g, 0], 0, _N - 1)
        a_w = (slot == tgt_g).astype(jnp.float32) * wmf[g, 0]
        aw_parts.append(a_w)

        awc = (ncol == tgt_g).astype(jnp.float32) * wmf[g, 0]
        umcol = umc_ref[g]
        ahc = umcol * (1.0 - awc)
        ones = jnp.ones((_N, 1), jnp.float32)
        a3 = jnp.concatenate([ones, ahc, awc], axis=1)
        v3 = jnp.concatenate(
            [e, 0.5 * uvec_ref[g:g + 1, :] - 0.5 * e,
             wvec_ref[g:g + 1, :] - e], axis=0)
        out_ref[g] = jnp.dot(a3, v3, preferred_element_type=jnp.float32)

    a_w_all = jnp.concatenate(aw_parts, axis=0)
    orf = jnp.maximum(umf, rmf)
    s_alive = jnp.sum(orf) + jnp.sum(a_w_all * (1.0 - orf))
    s_upd = jnp.sum(umf)
    s_fgt = jnp.sum(fmf)
    s_wm = jnp.sum(wmf)

    lane = jax.lax.broadcasted_iota(jnp.int32, (1, 128), 1)
    partial = (jnp.where(lane == 0, s_alive, 0.0)
               + jnp.where(lane == 2, s_wm, 0.0)
               + jnp.where(lane == 3, s_upd, 0.0)
               + jnp.where(lane == 4, s_fgt, 0.0))

    @pl.when(i == 0)
    def _init():
        stats_ref[...] = jnp.zeros_like(stats_ref)

    stats_ref[...] += partial

    @pl.when(i == nsteps - 1)
    def _finalize():
        scale = (jnp.where(lane == 0, 1.0 / (_B * _N), 0.0)
                 + jnp.where(lane == 2, 1.0 / _B, 0.0)
                 + jnp.where(lane == 3, 1.0 / (_B * _N), 0.0)
                 + jnp.where(lane == 4, 1.0 / (_B * _N), 0.0))
        stats_ref[...] = stats_ref[...] * scale


def kernel(empty, update_vec, write_vec, retain_mask, update_mask,
           forget_mask, write_mask, overwrite_idx):
    e2d = empty.reshape(1, _D).astype(jnp.float32)
    um2 = update_mask.reshape(_B * _NL, 128)
    rm2 = retain_mask.reshape(_B * _NL, 128)
    fm2 = forget_mask.reshape(_B * _NL, 128)
    umc = update_mask.astype(jnp.float32).reshape(_B, _N, 1)
    uv2 = update_vec.astype(jnp.float32).reshape(_B // _BG, _BG, _D)
    wv2 = write_vec.astype(jnp.float32).reshape(_B // _BG, _BG, _D)
    idx2 = overwrite_idx.astype(jnp.int32).reshape(_B // _BG, _BG, 1)
    wm2 = write_mask.astype(jnp.float32).reshape(_B // _BG, _BG, 1)

    bg16 = _BG * _NL
    mem, stats = pl.pallas_call(
        _slots_kernel,
        grid=(_B // _BG,),
        in_specs=[
            pl.BlockSpec((bg16, 128), lambda i: (i, 0)),
            pl.BlockSpec((bg16, 128), lambda i: (i, 0)),
            pl.BlockSpec((bg16, 128), lambda i: (i, 0)),
            pl.BlockSpec((_BG, _N, 1), lambda i: (i, 0, 0)),
            pl.BlockSpec((None, _BG, _D), lambda i: (i, 0, 0)),
            pl.BlockSpec((None, _BG, _D), lambda i: (i, 0, 0)),
            pl.BlockSpec((1, _D), lambda i: (0, 0)),
            pl.BlockSpec((None, _BG, 1), lambda i: (i, 0, 0)),
            pl.BlockSpec((None, _BG, 1), lambda i: (i, 0, 0)),
        ],
        out_specs=[
            pl.BlockSpec((_BG, _N, _D), lambda i: (i, 0, 0)),
            pl.BlockSpec((1, 128), lambda i: (0, 0)),
        ],
        out_shape=[
            jax.ShapeDtypeStruct((_B, _N, _D), jnp.float32),
            jax.ShapeDtypeStruct((1, 128), jnp.float32),
        ],
    )(um2, rm2, fm2, umc, uv2, wv2, e2d, idx2, wm2)

    return (mem, stats[0, 0], stats[0, 1], stats[0, 2],
            stats[0, 3], stats[0, 4])


# speedup vs baseline: 1.2595x; 1.2595x over previous
import numpy as np
import jax
import jax.numpy as jnp
from jax.experimental import pallas as pl

_B, _N, _D = 256, 2048, 64
_NL = _N // 128
_BG = 2
_NC = 256

_E1 = np.asarray(
    np.arange(_N)[:, None] // 128 == np.arange(_NL)[None, :], np.float32)
_M1 = np.asarray(
    np.arange(_N)[:, None] % 128 == np.arange(128)[None, :], np.float32)


def _slots_kernel(um_ref, rm_ref, fm_ref, uvec_ref, wvec_ref, e_ref,
                  idx_ref, wm_ref, e1_ref, m1_ref, out_ref, stats_ref):
    i = pl.program_id(0)
    nsteps = pl.num_programs(0)

    e = e_ref[...]
    umf = um_ref[...].astype(jnp.float32)
    rmf = rm_ref[...].astype(jnp.float32)
    fmf = fm_ref[...].astype(jnp.float32)
    wmf = wm_ref[...]
    e1 = e1_ref[...]
    m1 = m1_ref[...]

    row = jax.lax.broadcasted_iota(jnp.int32, (_NL, 128), 0)
    col = jax.lax.broadcasted_iota(jnp.int32, (_NL, 128), 1)
    slot = row * 128 + col

    aw_parts = []
    for g in range(_BG):
        tgt_g = jnp.clip(idx_ref[g, 0], 0, _N - 1)
        a_w = (slot == tgt_g).astype(jnp.float32) * wmf[g, 0]
        aw_parts.append(a_w)
        c_h = umf[g * _NL:(g + 1) * _NL, :] * (1.0 - a_w)

        c2 = jnp.concatenate([c_h, a_w], axis=1).astype(jnp.bfloat16)
        vh = jnp.broadcast_to(0.5 * uvec_ref[g:g + 1, :] - 0.5 * e,
                              (128, _D))
        vw = jnp.broadcast_to(wvec_ref[g:g + 1, :] - e, (128, _D))
        for ns in range(0, _N, _NC):
            t2 = jnp.dot(e1[ns:ns + _NC, :], c2,
                         preferred_element_type=jnp.float32)
            m1c = m1[ns:ns + _NC, :]
            mh = t2[:, :128] * m1c
            mw = t2[:, 128:] * m1c
            ph = jnp.dot(mh, vh, preferred_element_type=jnp.float32)
            pw = jnp.dot(mw, vw, preferred_element_type=jnp.float32)
            out_ref[g, ns:ns + _NC, :] = e + ph + pw

    a_w_all = jnp.concatenate(aw_parts, axis=0)
    orf = jnp.maximum(umf, rmf)
    s_alive = jnp.sum(orf) + jnp.sum(a_w_all * (1.0 - orf))
    s_upd = jnp.sum(umf)
    s_fgt = jnp.sum(fmf)
    s_wm = jnp.sum(wmf)

    lane = jax.lax.broadcasted_iota(jnp.int32, (1, 128), 1)
    partial = (jnp.where(lane == 0, s_alive, 0.0)
               + jnp.where(lane == 2, s_wm, 0.0)
               + jnp.where(lane == 3, s_upd, 0.0)
               + jnp.where(lane == 4, s_fgt, 0.0))

    @pl.when(i == 0)
    def _init():
        stats_ref[...] = jnp.zeros_like(stats_ref)

    stats_ref[...] += partial

    @pl.when(i == nsteps - 1)
    def _finalize():
        scale = (jnp.where(lane == 0, 1.0 / (_B * _N), 0.0)
                 + jnp.where(lane == 2, 1.0 / _B, 0.0)
                 + jnp.where(lane == 3, 1.0 / (_B * _N), 0.0)
                 + jnp.where(lane == 4, 1.0 / (_B * _N), 0.0))
        stats_ref[...] = stats_ref[...] * scale


def kernel(empty, update_vec, write_vec, retain_mask, update_mask,
           forget_mask, write_mask, overwrite_idx):
    e2d = empty.reshape(1, _D).astype(jnp.float32)
    um2 = update_mask.reshape(_B * _NL, 128)
    rm2 = retain_mask.reshape(_B * _NL, 128)
    fm2 = forget_mask.reshape(_B * _NL, 128)
    uv2 = update_vec.astype(jnp.float32).reshape(_B // _BG, _BG, _D)
    wv2 = write_vec.astype(jnp.float32).reshape(_B // _BG, _BG, _D)
    idx2 = overwrite_idx.astype(jnp.int32).reshape(_B // _BG, _BG, 1)
    wm2 = write_mask.astype(jnp.float32).reshape(_B // _BG, _BG, 1)

    bg16 = _BG * _NL
    mem, stats = pl.pallas_call(
        _slots_kernel,
        grid=(_B // _BG,),
        in_specs=[
            pl.BlockSpec((bg16, 128), lambda i: (i, 0)),
            pl.BlockSpec((bg16, 128), lambda i: (i, 0)),
            pl.BlockSpec((bg16, 128), lambda i: (i, 0)),
            pl.BlockSpec((None, _BG, _D), lambda i: (i, 0, 0)),
            pl.BlockSpec((None, _BG, _D), lambda i: (i, 0, 0)),
            pl.BlockSpec((1, _D), lambda i: (0, 0)),
            pl.BlockSpec((None, _BG, 1), lambda i: (i, 0, 0)),
            pl.BlockSpec((None, _BG, 1), lambda i: (i, 0, 0)),
            pl.BlockSpec((_N, _NL), lambda i: (0, 0)),
            pl.BlockSpec((_N, 128), lambda i: (0, 0)),
        ],
        out_specs=[
            pl.BlockSpec((_BG, _N, _D), lambda i: (i, 0, 0)),
            pl.BlockSpec((1, 128), lambda i: (0, 0)),
        ],
        out_shape=[
            jax.ShapeDtypeStruct((_B, _N, _D), jnp.float32),
            jax.ShapeDtypeStruct((1, 128), jnp.float32),
        ],
    )(um2, rm2, fm2, uv2, wv2, e2d, idx2, wm2,
      jnp.asarray(_E1, jnp.bfloat16), jnp.asarray(_M1))

    return (mem, stats[0, 0], stats[0, 1], stats[0, 2],
            stats[0, 3], stats[0, 4])


# speedup vs baseline: 1.2673x; 1.0062x over previous
import numpy as np
import jax
import jax.numpy as jnp
from jax.experimental import pallas as pl
from jax.experimental.pallas import tpu as pltpu

_B, _N, _D = 256, 2048, 64
_NL = _N // 128
_BG = 2
_NC = 512
_NBUF = 4

_E1 = np.asarray(
    np.arange(_N)[:, None] // 128 == np.arange(_NL)[None, :], np.float32)
_M1 = np.asarray(
    np.arange(_N)[:, None] % 128 == np.arange(128)[None, :], np.float32)


def _slots_kernel(um_ref, rm_ref, fm_ref, uvec_ref, wvec_ref, e_ref,
                  idx_ref, wm_ref, e1_ref, m1_ref, mem_ref, stats_ref,
                  obuf_ref, osem_ref):
    i = pl.program_id(0)
    nsteps = pl.num_programs(0)
    ring = jax.lax.rem(i, _NBUF)

    @pl.when(i >= _NBUF)
    def _recycle():
        pltpu.make_async_copy(
            obuf_ref.at[ring], mem_ref.at[pl.ds((i - _NBUF) * _BG, _BG)],
            osem_ref.at[ring]).wait()

    e = e_ref[...]
    umf = um_ref[...].astype(jnp.float32)
    rmf = rm_ref[...].astype(jnp.float32)
    fmf = fm_ref[...].astype(jnp.float32)
    wmf = wm_ref[...]
    e1 = e1_ref[...]
    m1 = m1_ref[...]

    row = jax.lax.broadcasted_iota(jnp.int32, (_NL, 128), 0)
    col = jax.lax.broadcasted_iota(jnp.int32, (_NL, 128), 1)
    slotid = row * 128 + col

    aw_parts = []
    for g in range(_BG):
        tgt_g = jnp.clip(idx_ref[g, 0], 0, _N - 1)
        a_w = (slotid == tgt_g).astype(jnp.float32) * wmf[g, 0]
        aw_parts.append(a_w)
        c_h = umf[g * _NL:(g + 1) * _NL, :] * (1.0 - a_w)

        c2 = jnp.concatenate([c_h, a_w], axis=1).astype(jnp.bfloat16)
        vh = jnp.broadcast_to(0.5 * uvec_ref[g:g + 1, :] - 0.5 * e,
                              (128, _D))
        vw = jnp.broadcast_to(wvec_ref[g:g + 1, :] - e, (128, _D))
        for ns in range(0, _N, _NC):
            t2 = jnp.dot(e1[ns:ns + _NC, :], c2,
                         preferred_element_type=jnp.float32)
            m1c = m1[ns:ns + _NC, :]
            mh = t2[:, :128] * m1c
            mw = t2[:, 128:] * m1c
            ph = jnp.dot(mh, vh, preferred_element_type=jnp.float32)
            pw = jnp.dot(mw, vw, preferred_element_type=jnp.float32)
            obuf_ref[ring, g, ns:ns + _NC, :] = e + ph + pw

    pltpu.make_async_copy(
        obuf_ref.at[ring], mem_ref.at[pl.ds(i * _BG, _BG)],
        osem_ref.at[ring]).start()

    a_w_all = jnp.concatenate(aw_parts, axis=0)
    orf = jnp.maximum(umf, rmf)
    s_alive = jnp.sum(orf) + jnp.sum(a_w_all * (1.0 - orf))
    s_upd = jnp.sum(umf)
    s_fgt = jnp.sum(fmf)
    s_wm = jnp.sum(wmf)

    lane = jax.lax.broadcasted_iota(jnp.int32, (1, 128), 1)
    partial = (jnp.where(lane == 0, s_alive, 0.0)
               + jnp.where(lane == 2, s_wm, 0.0)
               + jnp.where(lane == 3, s_upd, 0.0)
               + jnp.where(lane == 4, s_fgt, 0.0))

    @pl.when(i == 0)
    def _init():
        stats_ref[...] = jnp.zeros_like(stats_ref)

    stats_ref[...] += partial

    @pl.when(i == nsteps - 1)
    def _finalize():
        scale = (jnp.where(lane == 0, 1.0 / (_B * _N), 0.0)
                 + jnp.where(lane == 2, 1.0 / _B, 0.0)
                 + jnp.where(lane == 3, 1.0 / (_B * _N), 0.0)
                 + jnp.where(lane == 4, 1.0 / (_B * _N), 0.0))
        stats_ref[...] = stats_ref[...] * scale
        for k in range(_NBUF):
            pltpu.make_async_copy(
                obuf_ref.at[k], mem_ref.at[pl.ds(0, _BG)],
                osem_ref.at[k]).wait()


def kernel(empty, update_vec, write_vec, retain_mask, update_mask,
           forget_mask, write_mask, overwrite_idx):
    e2d = empty.reshape(1, _D).astype(jnp.float32)
    um2 = update_mask.reshape(_B * _NL, 128)
    rm2 = retain_mask.reshape(_B * _NL, 128)
    fm2 = forget_mask.reshape(_B * _NL, 128)
    uv2 = update_vec.astype(jnp.float32).reshape(_B // _BG, _BG, _D)
    wv2 = write_vec.astype(jnp.float32).reshape(_B // _BG, _BG, _D)
    idx2 = overwrite_idx.astype(jnp.int32).reshape(_B // _BG, _BG, 1)
    wm2 = write_mask.astype(jnp.float32).reshape(_B // _BG, _BG, 1)

    bg16 = _BG * _NL
    mem, stats = pl.pallas_call(
        _slots_kernel,
        grid=(_B // _BG,),
        in_specs=[
            pl.BlockSpec((bg16, 128), lambda i: (i, 0)),
            pl.BlockSpec((bg16, 128), lambda i: (i, 0)),
            pl.BlockSpec((bg16, 128), lambda i: (i, 0)),
            pl.BlockSpec((None, _BG, _D), lambda i: (i, 0, 0)),
            pl.BlockSpec((None, _BG, _D), lambda i: (i, 0, 0)),
            pl.BlockSpec((1, _D), lambda i: (0, 0)),
            pl.BlockSpec((None, _BG, 1), lambda i: (i, 0, 0)),
            pl.BlockSpec((None, _BG, 1), lambda i: (i, 0, 0)),
            pl.BlockSpec((_N, _NL), lambda i: (0, 0)),
            pl.BlockSpec((_N, 128), lambda i: (0, 0)),
        ],
        out_specs=[
            pl.BlockSpec(memory_space=pl.ANY),
            pl.BlockSpec((1, 128), lambda i: (0, 0)),
        ],
        out_shape=[
            jax.ShapeDtypeStruct((_B, _N, _D), jnp.float32),
            jax.ShapeDtypeStruct((1, 128), jnp.float32),
        ],
        scratch_shapes=[
            pltpu.VMEM((_NBUF, _BG, _N, _D), jnp.float32),
            pltpu.SemaphoreType.DMA((_NBUF,)),
        ],
    )(um2, rm2, fm2, uv2, wv2, e2d, idx2, wm2,
      jnp.asarray(_E1, jnp.bfloat16), jnp.asarray(_M1))

    return (mem, stats[0, 0], stats[0, 1], stats[0, 2],
            stats[0, 3], stats[0, 4])


# speedup vs baseline: 1.4026x; 1.1067x over previous
import numpy as np
import jax
import jax.numpy as jnp
from jax.experimental import pallas as pl

_B, _N, _D = 256, 2048, 64
_NL = _N // 128
_BG = 8
_NC = 256

_E1 = np.asarray(
    np.arange(_N)[:, None] // 128 == np.arange(_NL)[None, :], np.float32)
_M1 = np.asarray(
    np.arange(_N)[:, None] % 128 == np.arange(128)[None, :], np.float32)


def _slots_kernel(um_ref, rm_ref, fm_ref, uvec_ref, wvec_ref, e_ref,
                  idx_ref, wm_ref, e1_ref, m1_ref, out_ref, stats_ref):
    i = pl.program_id(0)
    nsteps = pl.num_programs(0)

    e = e_ref[...]
    umf = um_ref[...].astype(jnp.float32)
    rmf = rm_ref[...].astype(jnp.float32)
    fmf = fm_ref[...].astype(jnp.float32)
    wmf = wm_ref[...]
    e1 = e1_ref[...]
    m1 = m1_ref[...]

    row = jax.lax.broadcasted_iota(jnp.int32, (_NL, 128), 0)
    col = jax.lax.broadcasted_iota(jnp.int32, (_NL, 128), 1)
    slot = row * 128 + col

    aw_parts = []
    for g in range(_BG):
        tgt_g = jnp.clip(idx_ref[g, 0], 0, _N - 1)
        a_w = (slot == tgt_g).astype(jnp.float32) * wmf[g, 0]
        aw_parts.append(a_w)
        c_h = umf[g * _NL:(g + 1) * _NL, :] * (1.0 - a_w)

        c2 = jnp.concatenate([c_h, a_w], axis=1).astype(jnp.bfloat16)
        vh = jnp.broadcast_to(0.5 * uvec_ref[g:g + 1, :] - 0.5 * e,
                              (128, _D))
        vw = jnp.broadcast_to(wvec_ref[g:g + 1, :] - e, (128, _D))
        for ns in range(0, _N, _NC):
            t2 = jnp.dot(e1[ns:ns + _NC, :], c2,
                         preferred_element_type=jnp.float32)
            m1c = m1[ns:ns + _NC, :]
            mh = t2[:, :128] * m1c
            mw = t2[:, 128:] * m1c
            ph = jnp.dot(mh, vh, preferred_element_type=jnp.float32)
            pw = jnp.dot(mw, vw, preferred_element_type=jnp.float32)
            out_ref[g, ns:ns + _NC, :] = e + ph + pw

    a_w_all = jnp.concatenate(aw_parts, axis=0)
    orf = jnp.maximum(umf, rmf)
    s_alive = jnp.sum(orf) + jnp.sum(a_w_all * (1.0 - orf))
    s_upd = jnp.sum(umf)
    s_fgt = jnp.sum(fmf)
    s_wm = jnp.sum(wmf)

    lane = jax.lax.broadcasted_iota(jnp.int32, (1, 128), 1)
    partial = (jnp.where(lane == 0, s_alive, 0.0)
               + jnp.where(lane == 2, s_wm, 0.0)
               + jnp.where(lane == 3, s_upd, 0.0)
               + jnp.where(lane == 4, s_fgt, 0.0))

    @pl.when(i == 0)
    def _init():
        stats_ref[...] = jnp.zeros_like(stats_ref)

    stats_ref[...] += partial

    @pl.when(i == nsteps - 1)
    def _finalize():
        scale = (jnp.where(lane == 0, 1.0 / (_B * _N), 0.0)
                 + jnp.where(lane == 2, 1.0 / _B, 0.0)
                 + jnp.where(lane == 3, 1.0 / (_B * _N), 0.0)
                 + jnp.where(lane == 4, 1.0 / (_B * _N), 0.0))
        stats_ref[...] = stats_ref[...] * scale


def kernel(empty, update_vec, write_vec, retain_mask, update_mask,
           forget_mask, write_mask, overwrite_idx):
    e2d = empty.reshape(1, _D).astype(jnp.float32)
    um2 = update_mask.reshape(_B * _NL, 128)
    rm2 = retain_mask.reshape(_B * _NL, 128)
    fm2 = forget_mask.reshape(_B * _NL, 128)
    uv2 = update_vec.astype(jnp.float32).reshape(_B // _BG, _BG, _D)
    wv2 = write_vec.astype(jnp.float32).reshape(_B // _BG, _BG, _D)
    idx2 = overwrite_idx.astype(jnp.int32).reshape(_B // _BG, _BG, 1)
    wm2 = write_mask.astype(jnp.float32).reshape(_B // _BG, _BG, 1)

    bg16 = _BG * _NL
    mem, stats = pl.pallas_call(
        _slots_kernel,
        grid=(_B // _BG,),
        in_specs=[
            pl.BlockSpec((bg16, 128), lambda i: (i, 0)),
            pl.BlockSpec((bg16, 128), lambda i: (i, 0)),
            pl.BlockSpec((bg16, 128), lambda i: (i, 0)),
            pl.BlockSpec((None, _BG, _D), lambda i: (i, 0, 0)),
            pl.BlockSpec((None, _BG, _D), lambda i: (i, 0, 0)),
            pl.BlockSpec((1, _D), lambda i: (0, 0)),
            pl.BlockSpec((None, _BG, 1), lambda i: (i, 0, 0)),
            pl.BlockSpec((None, _BG, 1), lambda i: (i, 0, 0)),
            pl.BlockSpec((_N, _NL), lambda i: (0, 0)),
            pl.BlockSpec((_N, 128), lambda i: (0, 0)),
        ],
        out_specs=[
            pl.BlockSpec((_BG, _N, _D), lambda i: (i, 0, 0)),
            pl.BlockSpec((1, 128), lambda i: (0, 0)),
        ],
        out_shape=[
            jax.ShapeDtypeStruct((_B, _N, _D), jnp.float32),
            jax.ShapeDtypeStruct((1, 128), jnp.float32),
        ],
    )(um2, rm2, fm2, uv2, wv2, e2d, idx2, wm2,
      jnp.asarray(_E1, jnp.bfloat16), jnp.asarray(_M1))

    return (mem, stats[0, 0], stats[0, 1], stats[0, 2],
            stats[0, 3], stats[0, 4])


# speedup vs baseline: 1.4186x; 1.0114x over previous
import numpy as np
import jax
import jax.numpy as jnp
from jax.experimental import pallas as pl

_B, _N, _D = 256, 2048, 64
_NL = _N // 128
_BG = 16
_NC = 256

_E1 = np.asarray(
    np.arange(_N)[:, None] // 128 == np.arange(_NL)[None, :], np.float32)
_M1 = np.asarray(
    np.arange(_N)[:, None] % 128 == np.arange(128)[None, :], np.float32)


def _slots_kernel(um_ref, rm_ref, fm_ref, uvec_ref, wvec_ref, e_ref,
                  idx_ref, wm_ref, e1_ref, m1_ref, out_ref, stats_ref):
    i = pl.program_id(0)
    nsteps = pl.num_programs(0)

    e = e_ref[...]
    umf = um_ref[...].astype(jnp.float32)
    rmf = rm_ref[...].astype(jnp.float32)
    fmf = fm_ref[...].astype(jnp.float32)
    wmf = wm_ref[...]
    e1 = e1_ref[...]
    m1 = m1_ref[...]

    row = jax.lax.broadcasted_iota(jnp.int32, (_NL, 128), 0)
    col = jax.lax.broadcasted_iota(jnp.int32, (_NL, 128), 1)
    slot = row * 128 + col

    aw_parts = []
    for g in range(_BG):
        tgt_g = jnp.clip(idx_ref[g, 0], 0, _N - 1)
        a_w = (slot == tgt_g).astype(jnp.float32) * wmf[g, 0]
        aw_parts.append(a_w)
        c_h = umf[g * _NL:(g + 1) * _NL, :] * (1.0 - a_w)

        c2 = jnp.concatenate([c_h, a_w], axis=1).astype(jnp.bfloat16)
        vh = jnp.broadcast_to(0.5 * uvec_ref[g:g + 1, :] - 0.5 * e,
                              (128, _D))
        vw = jnp.broadcast_to(wvec_ref[g:g + 1, :] - e, (128, _D))
        for ns in range(0, _N, _NC):
            t2 = jnp.dot(e1[ns:ns + _NC, :], c2,
                         preferred_element_type=jnp.float32)
            m1c = m1[ns:ns + _NC, :]
            mh = t2[:, :128] * m1c
            mw = t2[:, 128:] * m1c
            ph = jnp.dot(mh, vh, preferred_element_type=jnp.float32)
            pw = jnp.dot(mw, vw, preferred_element_type=jnp.float32)
            out_ref[g, ns:ns + _NC, :] = e + ph + pw

    a_w_all = jnp.concatenate(aw_parts, axis=0)
    orf = jnp.maximum(umf, rmf)
    s_alive = jnp.sum(orf) + jnp.sum(a_w_all * (1.0 - orf))
    s_upd = jnp.sum(umf)
    s_fgt = jnp.sum(fmf)
    s_wm = jnp.sum(wmf)

    lane = jax.lax.broadcasted_iota(jnp.int32, (1, 128), 1)
    partial = (jnp.where(lane == 0, s_alive, 0.0)
               + jnp.where(lane == 2, s_wm, 0.0)
               + jnp.where(lane == 3, s_upd, 0.0)
               + jnp.where(lane == 4, s_fgt, 0.0))

    @pl.when(i == 0)
    def _init():
        stats_ref[...] = jnp.zeros_like(stats_ref)

    stats_ref[...] += partial

    @pl.when(i == nsteps - 1)
    def _finalize():
        scale = (jnp.where(lane == 0, 1.0 / (_B * _N), 0.0)
                 + jnp.where(lane == 2, 1.0 / _B, 0.0)
                 + jnp.where(lane == 3, 1.0 / (_B * _N), 0.0)
                 + jnp.where(lane == 4, 1.0 / (_B * _N), 0.0))
        stats_ref[...] = stats_ref[...] * scale


def kernel(empty, update_vec, write_vec, retain_mask, update_mask,
           forget_mask, write_mask, overwrite_idx):
    e2d = empty.reshape(1, _D).astype(jnp.float32)
    um2 = update_mask.reshape(_B * _NL, 128)
    rm2 = retain_mask.reshape(_B * _NL, 128)
    fm2 = forget_mask.reshape(_B * _NL, 128)
    uv2 = update_vec.astype(jnp.float32).reshape(_B // _BG, _BG, _D)
    wv2 = write_vec.astype(jnp.float32).reshape(_B // _BG, _BG, _D)
    idx2 = overwrite_idx.astype(jnp.int32).reshape(_B // _BG, _BG, 1)
    wm2 = write_mask.astype(jnp.float32).reshape(_B // _BG, _BG, 1)

    bg16 = _BG * _NL
    mem, stats = pl.pallas_call(
        _slots_kernel,
        grid=(_B // _BG,),
        in_specs=[
            pl.BlockSpec((bg16, 128), lambda i: (i, 0)),
            pl.BlockSpec((bg16, 128), lambda i: (i, 0)),
            pl.BlockSpec((bg16, 128), lambda i: (i, 0)),
            pl.BlockSpec((None, _BG, _D), lambda i: (i, 0, 0)),
            pl.BlockSpec((None, _BG, _D), lambda i: (i, 0, 0)),
            pl.BlockSpec((1, _D), lambda i: (0, 0)),
            pl.BlockSpec((None, _BG, 1), lambda i: (i, 0, 0)),
            pl.BlockSpec((None, _BG, 1), lambda i: (i, 0, 0)),
            pl.BlockSpec((_N, _NL), lambda i: (0, 0)),
            pl.BlockSpec((_N, 128), lambda i: (0, 0)),
        ],
        out_specs=[
            pl.BlockSpec((_BG, _N, _D), lambda i: (i, 0, 0)),
            pl.BlockSpec((1, 128), lambda i: (0, 0)),
        ],
        out_shape=[
            jax.ShapeDtypeStruct((_B, _N, _D), jnp.float32),
            jax.ShapeDtypeStruct((1, 128), jnp.float32),
        ],
    )(um2, rm2, fm2, uv2, wv2, e2d, idx2, wm2,
      jnp.asarray(_E1, jnp.bfloat16), jnp.asarray(_M1))

    return (mem, stats[0, 0], stats[0, 1], stats[0, 2],
            stats[0, 3], stats[0, 4])
